# baseline (speedup 1.0000x reference)
import jax
import jax.numpy as jnp
from jax.experimental import pallas as pl
from jax.experimental.pallas import tpu as pltpu

F = 128
W = 5
C = 16
D = C * F + C * W + F * F + W * W
BLK = 1024
NBLK = (D + BLK - 1) // BLK
NGRID = (NBLK + 3) // 4
DPAD = 4 * NGRID * BLK


def _bc(v, m):
    return jnp.broadcast_to(v[m:m + 1, :], (W, 1))


_XOFF = 0
_WTI = _XOFF + F * W
_WIR, _WIZ, _WIN = _WTI + 25, _WTI + 50, _WTI + 75
_WHR, _WHZ, _WHN = _WTI + 100, _WTI + 125, _WTI + 150
_BTI = _WTI + 175
_BIR, _BIZ, _BIN = _BTI + 5, _BTI + 10, _BTI + 15
_BHR, _BHZ, _BHN = _BTI + 20, _BTI + 25, _BTI + 30
_CP = _BTI + 35 + 6


GSTEP = F // (NGRID - 1)


def _sig(a):
    return 0.5 * jnp.tanh(0.5 * a) + 0.5


def _body(x, colpack, fft, Wfu, bfu,
          Wta, Wt, Wsa, Ws, wf_a, wf_b, wf_c, wf_d, bf,
          out_ref, cat_ref, acc_ref, vs_ref, h_ref, sv_ref):
    i = pl.program_id(0)

    @pl.when(i == 0)
    def _init():
        vt = jax.nn.sigmoid(jnp.sum(Wta[...], axis=0, keepdims=True))
        vs = jax.nn.sigmoid(jnp.sum(Wsa[...], axis=0, keepdims=True))

        vs_ref[...] = jax.nn.sigmoid(jax.lax.dot_general(
            Wsa[...], jnp.ones((F, 1), jnp.float32),
            (((0,), (0,)), ((), ()))))
        h_ref[...] = jnp.zeros((W, 1), jnp.float32)
        sv_ref[...] = jnp.zeros((W, 1), jnp.float32)

        base = W * C + F * C
        for w in range(W):
            cat_ref[0:1, base + w * W:base + (w + 1) * W] = vt
        base = base + W * W
        for u in range(F):
            cat_ref[0:1, base + u * F:base + (u + 1) * F] = vs
        cat_ref[0:1, D:DPAD] = jnp.zeros((1, DPAD - D), jnp.float32)
        acc_ref[...] = jnp.zeros_like(acc_ref)

    @pl.when(i < NGRID - 1)
    def _gru_chunk():
        cpc = lambda off: colpack[off:off + W, :]
        wti_m = [cpc(_WTI + W * m) for m in range(W)]
        wir_m = [cpc(_WIR + W * m) for m in range(W)]
        wiz_m = [cpc(_WIZ + W * m) for m in range(W)]
        win_m = [cpc(_WIN + W * m) for m in range(W)]
        whr_m = [cpc(_WHR + W * m) for m in range(W)]
        whz_m = [cpc(_WHZ + W * m) for m in range(W)]
        whn_m = [cpc(_WHN + W * m) for m in range(W)]
        btic = cpc(_BTI)
        birc, bizc, binc = cpc(_BIR), cpc(_BIZ), cpc(_BIN)
        bhrc, bhzc, bhnc = cpc(_BHR), cpc(_BHZ), cpc(_BHN)

        def step(t, carry):
            h, sv = carry
            xt = colpack[pl.ds(W * t, W), :]
            xt = jnp.where(jnp.isnan(xt), 0.0, xt)
            e = btic
            for m in range(W):
                e = e + wti_m[m] * _bc(xt, m)
            e = jax.nn.relu(e)
            gr, gz, gn = birc, bizc, binc
            for m in range(W):
                em = _bc(e, m)
                gr = gr + wir_m[m] * em
                gz = gz + wiz_m[m] * em
                gn = gn + win_m[m] * em
            hr, hz, hn_ = bhrc, bhzc, bhnc
            for m in range(W):
                hm = _bc(h, m)
                hr = hr + whr_m[m] * hm
                hz = hz + whz_m[m] * hm
                hn_ = hn_ + whn_m[m] * hm
            r = _sig(gr + hr)
            z = _sig(gz + hz)
            n = jnp.tanh(gn + r * hn_)
            hnew = (1.0 - z) * n + z * h
            vst = jnp.broadcast_to(vs_ref[pl.ds(t, 1), :], (W, 1))
            return hnew, sv + vst * hnew

        t0 = i * GSTEP
        h, sv = jax.lax.fori_loop(0, GSTEP,
                                  lambda k, c: step(t0 + k, c),
                                  (h_ref[...], sv_ref[...]))
        h_ref[...] = h
        sv_ref[...] = sv

    @pl.when(i == NGRID - 1)
    def _fill_sf():
        xc = jnp.where(jnp.isnan(x[...]), 0.0, x[...])
        freq = jax.nn.relu(jnp.dot(xc, Wfu[0:F, :])
                           + jnp.dot(fft[...], Wfu[F:2 * F, :])
                           + bfu[...][None, :])
        vt = jax.nn.sigmoid(jnp.sum(Wta[...], axis=0, keepdims=True))
        tf0 = jnp.dot(vt, jnp.dot(freq, Wt[...]))
        sf0 = jax.lax.dot_general(sv_ref[...], Ws[...],
                                  (((0,), (0,)), ((), ())))
        for w in range(W):
            cat_ref[0:1, w * C:(w + 1) * C] = tf0
        for u in range(F):
            cat_ref[0:1, W * C + u * C:W * C + (u + 1) * C] = sf0

    j = NGRID - 1 - i
    cat_a = cat_ref[0:1, pl.ds(4 * j * BLK, BLK)]
    cat_b = cat_ref[0:1, pl.ds((4 * j + 1) * BLK, BLK)]
    cat_c = cat_ref[0:1, pl.ds((4 * j + 2) * BLK, BLK)]
    cat_d = cat_ref[0:1, pl.ds((4 * j + 3) * BLK, BLK)]

    @pl.when(i > 0)
    def _full():
        acc_ref[...] += (jnp.dot(cat_a, wf_a[...],
                                 preferred_element_type=jnp.float32)
                         + jnp.dot(cat_b, wf_b[...],
                                   preferred_element_type=jnp.float32)
                         + jnp.dot(cat_c, wf_c[...],
                                   preferred_element_type=jnp.float32)
                         + jnp.dot(cat_d, wf_d[...],
                                   preferred_element_type=jnp.float32))

    @pl.when(i == 0)
    def _masked():
        nvalid = D - (NBLK - 1) * BLK
        rows = jax.lax.broadcasted_iota(jnp.int32, (BLK, 1), 0)
        wmask = jnp.where(rows < nvalid, wf_c[...], 0.0)
        acc_ref[...] += (jnp.dot(cat_a, wf_a[...],
                                 preferred_element_type=jnp.float32)
                         + jnp.dot(cat_b, wf_b[...],
                                   preferred_element_type=jnp.float32)
                         + jnp.dot(cat_c, wmask,
                                   preferred_element_type=jnp.float32))

    @pl.when(i == NGRID - 1)
    def _out():
        res = jax.nn.relu(acc_ref[...] + bf[...][None, :])
        for w in range(W):
            out_ref[w:w + 1, :] = res[0:1, w * F:(w + 1) * F]


@jax.jit
def kernel(x, fft, W_fuse, b_fuse, W_time, b_time, W_ih, W_hh, b_ih, b_hh,
           W_tadj, W_t, W_sadj, W_s, W_fcn, b_fcn):
    full = lambda shape: pl.BlockSpec(shape, lambda i: (0,) * len(shape))
    colpack = jnp.concatenate([
        x.T.ravel(),
        W_time.ravel(),
        W_ih.reshape(3, W, W).transpose(0, 2, 1).ravel(),
        W_hh.reshape(3, W, W).transpose(0, 2, 1).ravel(),
        b_time, b_ih, b_hh,
        jnp.zeros((_CP - _BTI - 35,), jnp.float32),
    ]).reshape(_CP, 1)
    out = pl.pallas_call(
        _body,
        grid=(NGRID,),
        in_specs=[
            full((W, F)),
            full((_CP, 1)),
            full((W, F)),
            full((2 * F, F)),
            pl.BlockSpec((F,), lambda i: (0,)),
            full((W, W)),
            full((F, C)),
            full((F, F)),
            full((W, C)),
            pl.BlockSpec((BLK, W * F),
                         lambda i: (4 * (NGRID - 1 - i), 0)),
            pl.BlockSpec((BLK, W * F),
                         lambda i: (4 * (NGRID - 1 - i) + 1, 0)),
            pl.BlockSpec((BLK, W * F),
                         lambda i: (4 * (NGRID - 1 - i) + 2, 0)),
            pl.BlockSpec((BLK, W * F),
                         lambda i: (jnp.minimum(4 * (NGRID - 1 - i) + 3,
                                                NBLK - 1), 0)),
            pl.BlockSpec((W * F,), lambda i: (0,)),
        ],
        out_specs=pl.BlockSpec((W, F), lambda i: (0, 0)),
        out_shape=jax.ShapeDtypeStruct((W, F), jnp.float32),
        scratch_shapes=[
            pltpu.VMEM((1, DPAD), jnp.float32),
            pltpu.VMEM((1, W * F), jnp.float32),
            pltpu.VMEM((F, 1), jnp.float32),
            pltpu.VMEM((W, 1), jnp.float32),
            pltpu.VMEM((W, 1), jnp.float32),
        ],
    )(x, colpack, fft, W_fuse, b_fuse,
      W_tadj, W_t, W_sadj, W_s, W_fcn, W_fcn, W_fcn, W_fcn, b_fcn)
    return out


# speedup vs baseline: 1.0364x; 1.0364x over previous
import jax
import jax.numpy as jnp
from jax.experimental import pallas as pl
from jax.experimental.pallas import tpu as pltpu

F = 128
W = 5
C = 16
D = C * F + C * W + F * F + W * W
BLK = 2048
NBLK = (D + BLK - 1) // BLK
DPAD = NBLK * BLK


def _bc(v, m):
    return jnp.broadcast_to(v[m:m + 1, :], (W, 1))


_XOFF = 0
_WTI = _XOFF + F * W
_WIR, _WIZ, _WIN = _WTI + 25, _WTI + 50, _WTI + 75
_WHR, _WHZ, _WHN = _WTI + 100, _WTI + 125, _WTI + 150
_BTI = _WTI + 175
_BIR, _BIZ, _BIN = _BTI + 5, _BTI + 10, _BTI + 15
_BHR, _BHZ, _BHN = _BTI + 20, _BTI + 25, _BTI + 30
_CP = _BTI + 35 + 6


NGRID = NBLK // 2
GSTEP = F // (NGRID - 1)


def _sig(a):
    return 0.5 * jnp.tanh(0.5 * a) + 0.5


def _body(x, colpack, fft, Wfu, bfu,
          Wta, Wt, Wsa, Ws, wf_a, wf_b, bf,
          out_ref, cat_ref, acc_ref, vs_ref, h_ref, sv_ref):
    i = pl.program_id(0)

    @pl.when(i == 0)
    def _init():
        vt = jax.nn.sigmoid(jnp.sum(Wta[...], axis=0, keepdims=True))
        vs = jax.nn.sigmoid(jnp.sum(Wsa[...], axis=0, keepdims=True))

        vs_ref[...] = jax.nn.sigmoid(jax.lax.dot_general(
            Wsa[...], jnp.ones((F, 1), jnp.float32),
            (((0,), (0,)), ((), ()))))
        h_ref[...] = jnp.zeros((W, 1), jnp.float32)
        sv_ref[...] = jnp.zeros((W, 1), jnp.float32)

        base = W * C + F * C
        for w in range(W):
            cat_ref[0:1, base + w * W:base + (w + 1) * W] = vt
        base = base + W * W
        for u in range(F):
            cat_ref[0:1, base + u * F:base + (u + 1) * F] = vs
        cat_ref[0:1, D:DPAD] = jnp.zeros((1, DPAD - D), jnp.float32)
        acc_ref[...] = jnp.zeros_like(acc_ref)

    @pl.when(i < NGRID - 1)
    def _gru_chunk():
        cpc = lambda off: colpack[off:off + W, :]
        wti_m = [cpc(_WTI + W * m) for m in range(W)]
        wir_m = [cpc(_WIR + W * m) for m in range(W)]
        wiz_m = [cpc(_WIZ + W * m) for m in range(W)]
        win_m = [cpc(_WIN + W * m) for m in range(W)]
        whr_m = [cpc(_WHR + W * m) for m in range(W)]
        whz_m = [cpc(_WHZ + W * m) for m in range(W)]
        whn_m = [cpc(_WHN + W * m) for m in range(W)]
        btic = cpc(_BTI)
        birc, bizc, binc = cpc(_BIR), cpc(_BIZ), cpc(_BIN)
        bhrc, bhzc, bhnc = cpc(_BHR), cpc(_BHZ), cpc(_BHN)

        def step(t, carry):
            h, sv = carry
            xt = colpack[pl.ds(W * t, W), :]
            xt = jnp.where(jnp.isnan(xt), 0.0, xt)
            e = btic
            for m in range(W):
                e = e + wti_m[m] * _bc(xt, m)
            e = jax.nn.relu(e)
            gr, gz, gn = birc, bizc, binc
            for m in range(W):
                em = _bc(e, m)
                gr = gr + wir_m[m] * em
                gz = gz + wiz_m[m] * em
                gn = gn + win_m[m] * em
            hr, hz, hn_ = bhrc, bhzc, bhnc
            for m in range(W):
                hm = _bc(h, m)
                hr = hr + whr_m[m] * hm
                hz = hz + whz_m[m] * hm
                hn_ = hn_ + whn_m[m] * hm
            r = _sig(gr + hr)
            z = _sig(gz + hz)
            n = jnp.tanh(gn + r * hn_)
            hnew = (1.0 - z) * n + z * h
            vst = jnp.broadcast_to(vs_ref[pl.ds(t, 1), :], (W, 1))
            return hnew, sv + vst * hnew

        t0 = i * GSTEP
        h, sv = jax.lax.fori_loop(0, GSTEP,
                                  lambda k, c: step(t0 + k, c),
                                  (h_ref[...], sv_ref[...]))
        h_ref[...] = h
        sv_ref[...] = sv

    @pl.when(i == NGRID - 1)
    def _fill_sf():
        xc = jnp.where(jnp.isnan(x[...]), 0.0, x[...])
        freq = jax.nn.relu(jnp.dot(xc, Wfu[0:F, :])
                           + jnp.dot(fft[...], Wfu[F:2 * F, :])
                           + bfu[...][None, :])
        vt = jax.nn.sigmoid(jnp.sum(Wta[...], axis=0, keepdims=True))
        tf0 = jnp.dot(vt, jnp.dot(freq, Wt[...]))
        sf0 = jax.lax.dot_general(sv_ref[...], Ws[...],
                                  (((0,), (0,)), ((), ())))
        for w in range(W):
            cat_ref[0:1, w * C:(w + 1) * C] = tf0
        for u in range(F):
            cat_ref[0:1, W * C + u * C:W * C + (u + 1) * C] = sf0

    j = NGRID - 1 - i
    cat_a = cat_ref[0:1, pl.ds(2 * j * BLK, BLK)]
    cat_b = cat_ref[0:1, pl.ds((2 * j + 1) * BLK, BLK)]

    @pl.when(i > 0)
    def _full():
        acc_ref[...] += (jnp.dot(cat_a, wf_a[...],
                                 preferred_element_type=jnp.float32)
                         + jnp.dot(cat_b, wf_b[...],
                                   preferred_element_type=jnp.float32))

    @pl.when(i == 0)
    def _masked():
        nvalid = D - (NBLK - 1) * BLK
        rows = jax.lax.broadcasted_iota(jnp.int32, (BLK, 1), 0)
        wmask = jnp.where(rows < nvalid, wf_b[...], 0.0)
        acc_ref[...] += (jnp.dot(cat_a, wf_a[...],
                                 preferred_element_type=jnp.float32)
                         + jnp.dot(cat_b, wmask,
                                   preferred_element_type=jnp.float32))

    @pl.when(i == NGRID - 1)
    def _out():
        res = jax.nn.relu(acc_ref[...] + bf[...][None, :])
        for w in range(W):
            out_ref[w:w + 1, :] = res[0:1, w * F:(w + 1) * F]


@jax.jit
def kernel(x, fft, W_fuse, b_fuse, W_time, b_time, W_ih, W_hh, b_ih, b_hh,
           W_tadj, W_t, W_sadj, W_s, W_fcn, b_fcn):
    full = lambda shape: pl.BlockSpec(shape, lambda i: (0,) * len(shape))
    colpack = jnp.concatenate([
        x.T.ravel(),
        W_time.ravel(),
        W_ih.reshape(3, W, W).transpose(0, 2, 1).ravel(),
        W_hh.reshape(3, W, W).transpose(0, 2, 1).ravel(),
        b_time, b_ih, b_hh,
        jnp.zeros((_CP - _BTI - 35,), jnp.float32),
    ]).reshape(_CP, 1)
    out = pl.pallas_call(
        _body,
        grid=(NGRID,),
        in_specs=[
            full((W, F)),
            full((_CP, 1)),
            full((W, F)),
            full((2 * F, F)),
            pl.BlockSpec((F,), lambda i: (0,)),
            full((W, W)),
            full((F, C)),
            full((F, F)),
            full((W, C)),
            pl.BlockSpec((BLK, W * F),
                         lambda i: (2 * (NGRID - 1 - i), 0)),
            pl.BlockSpec((BLK, W * F),
                         lambda i: (2 * (NGRID - 1 - i) + 1, 0)),
            pl.BlockSpec((W * F,), lambda i: (0,)),
        ],
        out_specs=pl.BlockSpec((W, F), lambda i: (0, 0)),
        out_shape=jax.ShapeDtypeStruct((W, F), jnp.float32),
        scratch_shapes=[
            pltpu.VMEM((1, DPAD), jnp.float32),
            pltpu.VMEM((1, W * F), jnp.float32),
            pltpu.VMEM((F, 1), jnp.float32),
            pltpu.VMEM((W, 1), jnp.float32),
            pltpu.VMEM((W, 1), jnp.float32),
        ],
    )(x, colpack, fft, W_fuse, b_fuse,
      W_tadj, W_t, W_sadj, W_s, W_fcn, W_fcn, b_fcn)
    return out
